# baseline (speedup 1.0000x reference)
import jax
import jax.numpy as jnp
from jax import lax
from jax.experimental import pallas as pl
from jax.experimental.pallas import tpu as pltpu
from jax.experimental.pallas import tpu_sc as plsc

B = 16384
D = 128
S = 25
C = 64
NC = 2
NS = 16
NW = NC * NS
PER_W = B // NW
K = 16
CHUNKS = PER_W // K
ROWS = K * S + K
GPC = 4
GLEN = ROWS // GPC
NLANE = 16
NVD = D // NLANE


def _sc_body(idx_hbm, feat_hbm, self_out, sum_out,
             idx0, idx1, rows0, rows1, sum0, sum1,
             sem0, sem1, osem0, osem1):
    cid = lax.axis_index("c")
    sid = lax.axis_index("s")
    wid = sid * NC + cid
    idxs = (idx0, idx1)
    rows = (rows0, rows1)
    sums = (sum0, sum1)
    sems = (sem0, sem1)
    osems = (osem0, osem1)

    def fire(c, b):
        t = wid * CHUNKS + c
        pltpu.sync_copy(idx_hbm.at[pl.ds(t * GPC, GPC)], idxs[b])
        for j in range(GPC):
            pltpu.async_copy(feat_hbm.at[idxs[b].at[j]],
                             rows[b].at[pl.ds(j * GLEN, GLEN)], sems[b])

    def drain(b):
        for j in range(GPC):
            pltpu.make_async_copy(feat_hbm.at[pl.ds(0, GLEN)],
                                  rows[b].at[pl.ds(j * GLEN, GLEN)],
                                  sems[b]).wait()

    def compute(c, b):
        rb = rows[b]
        sb = sums[b]

        @plsc.parallel_loop(0, K, unroll=2)
        def _elem(k):
            r0 = k * S
            acc = tuple(rb[r0, pl.ds(NLANE * d, NLANE)] for d in range(NVD))

            def _sbody(s2, a):
                return tuple(a[d] + rb[r0 + s2, pl.ds(NLANE * d, NLANE)]
                             for d in range(NVD))

            acc = lax.fori_loop(1, S, _sbody, acc, unroll=8)
            for d in range(NVD):
                sb[k, pl.ds(NLANE * d, NLANE)] = acc[d]

        base = (wid * CHUNKS + c) * K
        pltpu.async_copy(rb.at[pl.ds(K * S, K)], self_out.at[pl.ds(base, K)],
                         osems[b])
        pltpu.async_copy(sb, sum_out.at[pl.ds(base, K)], osems[b])

    def drain_out(b):
        pltpu.make_async_copy(feat_hbm.at[pl.ds(0, K)], sums[b],
                              osems[b]).wait()
        pltpu.make_async_copy(feat_hbm.at[pl.ds(0, K)],
                              rows[b].at[pl.ds(K * S, K)], osems[b]).wait()

    fire(0, 0)

    @pl.loop(0, CHUNKS, step=2)
    def _outer(cb):
        for b in range(2):
            c = cb + b

            @pl.when(c > 0)
            def _():
                drain_out(1 - b)

            @pl.when(c + 1 < CHUNKS)
            def _():
                fire(c + 1, 1 - b)

            drain(b)
            compute(c, b)

    drain_out(1)


def _sc_gather(idx_packed, features):
    f = pl.kernel(
        _sc_body,
        out_type=(jax.ShapeDtypeStruct((B, D), jnp.float32),
                  jax.ShapeDtypeStruct((B, D), jnp.float32)),
        mesh=plsc.VectorSubcoreMesh(core_axis_name="c", subcore_axis_name="s",
                                    num_cores=NC, num_subcores=NS),
        scratch_types=(
            pltpu.VMEM((GPC, GLEN), jnp.int32),
            pltpu.VMEM((GPC, GLEN), jnp.int32),
            pltpu.VMEM((ROWS, D), jnp.float32),
            pltpu.VMEM((ROWS, D), jnp.float32),
            pltpu.VMEM((K, D), jnp.float32),
            pltpu.VMEM((K, D), jnp.float32),
            pltpu.SemaphoreType.DMA,
            pltpu.SemaphoreType.DMA,
            pltpu.SemaphoreType.DMA,
            pltpu.SemaphoreType.DMA,
        ),
    )
    return f(idx_packed, features)


BM = 8192


def _tc_body(xs_ref, xm_ref, ws_ref, wn_ref, wc_ref, o_ref):
    h = jnp.dot(xs_ref[...], ws_ref[...], preferred_element_type=jnp.float32)
    h = h + jnp.dot(xm_ref[...], wn_ref[...], preferred_element_type=jnp.float32)
    h = jnp.maximum(h, 0.0)
    o_ref[...] = jnp.dot(h, wc_ref[...], preferred_element_type=jnp.float32)


def _tc_head(xs, xm, ws_t, wn_t, wc_t):
    return pl.pallas_call(
        _tc_body,
        grid=(B // BM,),
        in_specs=[
            pl.BlockSpec((BM, D), lambda i: (i, 0)),
            pl.BlockSpec((BM, D), lambda i: (i, 0)),
            pl.BlockSpec((D, D), lambda i: (0, 0)),
            pl.BlockSpec((D, D), lambda i: (0, 0)),
            pl.BlockSpec((D, C), lambda i: (0, 0)),
        ],
        out_specs=pl.BlockSpec((BM, C), lambda i: (i, 0)),
        out_shape=jax.ShapeDtypeStruct((B, C), jnp.float32),
    )(xs, xm, ws_t, wn_t, wc_t)


def kernel(nodes, neigh_idx, features, W_enc, weight):
    idx_packed = jnp.concatenate(
        [neigh_idx.reshape(B // K, K * S), nodes.reshape(B // K, K)], axis=1
    ).reshape(-1, GLEN)
    self_out, sum_out = _sc_gather(idx_packed, features)
    ws_t = W_enc[:, :D].T
    wn_t = W_enc[:, D:].T * jnp.float32(1.0 / S)
    wc_t = weight.T
    return _tc_head(self_out, sum_out, ws_t, wn_t, wc_t)
